# baseline (speedup 1.0000x reference)
import jax
import jax.numpy as jnp
from jax.experimental import pallas as pl
from jax.experimental.pallas import tpu as pltpu
from jax.experimental.pallas import tpu_sc as plsc

VOCAB = 100000
HIDDEN = 128
BASIN = 64

_VOCAB_BLOCK = 2000
_GATHER_WINDOW = 128


def _proj_softmax_body(w_ref, tp_ref, out_ref):
    logits = jax.lax.dot_general(
        tp_ref[...], w_ref[...],
        (((1,), (1,)), ((), ())),
        preferred_element_type=jnp.float32,
    )
    m = jnp.max(logits, axis=-1, keepdims=True)
    e = jnp.exp(logits - m)
    sm = e / jnp.sum(e, axis=-1, keepdims=True)
    out_ref[:, :BASIN] = sm
    out_ref[:, BASIN:] = jnp.zeros_like(sm)


def _project_softmax_table(token_params, basin_proj_w):
    grid = VOCAB // _VOCAB_BLOCK
    return pl.pallas_call(
        _proj_softmax_body,
        grid=(grid,),
        in_specs=[
            pl.BlockSpec((BASIN, HIDDEN), lambda i: (0, 0)),
            pl.BlockSpec((_VOCAB_BLOCK, HIDDEN), lambda i: (i, 0)),
        ],
        out_specs=pl.BlockSpec((_VOCAB_BLOCK, 2 * BASIN), lambda i: (i, 0)),
        out_shape=jax.ShapeDtypeStruct((VOCAB, 2 * BASIN), jnp.float32),
    )(basin_proj_w, token_params)


def _sc_gather(table, flat_ids):
    num_indices = flat_ids.shape[0]
    row_w = table.shape[1]
    ids2d = flat_ids.reshape(1, num_indices)
    mesh = plsc.VectorSubcoreMesh(core_axis_name="core",
                                  subcore_axis_name="subcore")

    @pl.kernel(
        out_type=jax.ShapeDtypeStruct((num_indices, row_w), jnp.float32),
        mesh=mesh,
    )
    def gather_kernel(table_hbm, ids_hbm, out_hbm):
        def body(ids_vmem, out_vmem):
            pltpu.sync_copy(table_hbm.at[ids_vmem.at[0]], out_vmem)

        pltpu.emit_pipeline(
            body,
            grid=(num_indices // _GATHER_WINDOW,),
            in_specs=[pl.BlockSpec((1, _GATHER_WINDOW),
                                   index_map=lambda i: (0, i))],
            out_specs=[pl.BlockSpec((_GATHER_WINDOW, row_w),
                                    index_map=lambda i: (i, 0))],
            core_axis_name=("core", "subcore"),
            dimension_semantics=(pltpu.PARALLEL,),
        )(ids_hbm, out_hbm)

    return gather_kernel(table, ids2d)


_N_CHUNKS = 4


@jax.jit
def kernel(token_ids, token_params, basin_proj_w):
    B, T = token_ids.shape
    table = _project_softmax_table(token_params, basin_proj_w)
    flat = token_ids.reshape(B * T).astype(jnp.int32)
    chunk = flat.shape[0] // _N_CHUNKS
    parts = []
    for c in range(_N_CHUNKS):
        g = _sc_gather(table, flat[c * chunk:(c + 1) * chunk])
        parts.append(g[:, :BASIN])
    out = jnp.concatenate(parts, axis=0)
    return out.reshape(B, T, BASIN)


# speedup vs baseline: 1.7432x; 1.7432x over previous
import jax
import jax.numpy as jnp
from jax.experimental import pallas as pl
from jax.experimental.pallas import tpu as pltpu
from jax.experimental.pallas import tpu_sc as plsc

VOCAB = 100000
HIDDEN = 128
BASIN = 64

_VOCAB_BLOCK = 2000
_GATHER_WINDOW = 256


def _proj_softmax_body(w_ref, tp_ref, out_ref):
    logits = jax.lax.dot_general(
        tp_ref[...], w_ref[...],
        (((1,), (1,)), ((), ())),
        preferred_element_type=jnp.float32,
    )
    m = jnp.max(logits, axis=-1, keepdims=True)
    e = jnp.exp(logits - m)
    sm = e / jnp.sum(e, axis=-1, keepdims=True)
    out_ref[:, :BASIN] = sm
    out_ref[:, BASIN:] = jnp.zeros_like(sm)


def _project_softmax_table(token_params, basin_proj_w):
    grid = VOCAB // _VOCAB_BLOCK
    return pl.pallas_call(
        _proj_softmax_body,
        grid=(grid,),
        in_specs=[
            pl.BlockSpec((BASIN, HIDDEN), lambda i: (0, 0)),
            pl.BlockSpec((_VOCAB_BLOCK, HIDDEN), lambda i: (i, 0)),
        ],
        out_specs=pl.BlockSpec((_VOCAB_BLOCK, 2 * BASIN), lambda i: (i, 0)),
        out_shape=jax.ShapeDtypeStruct((VOCAB, 2 * BASIN), jnp.float32),
    )(basin_proj_w, token_params)


def _sc_gather(table, flat_ids):
    num_indices = flat_ids.shape[0]
    row_w = table.shape[1]
    ids2d = flat_ids.reshape(1, num_indices)
    mesh = plsc.VectorSubcoreMesh(core_axis_name="core",
                                  subcore_axis_name="subcore")

    @pl.kernel(
        out_type=jax.ShapeDtypeStruct((num_indices, row_w), jnp.float32),
        mesh=mesh,
    )
    def gather_kernel(table_hbm, ids_hbm, out_hbm):
        def body(ids_vmem, out_vmem):
            pltpu.sync_copy(table_hbm.at[ids_vmem.at[0]], out_vmem)

        pltpu.emit_pipeline(
            body,
            grid=(num_indices // _GATHER_WINDOW,),
            in_specs=[pl.BlockSpec((1, _GATHER_WINDOW),
                                   index_map=lambda i: (0, i))],
            out_specs=[pl.BlockSpec((_GATHER_WINDOW, row_w),
                                    index_map=lambda i: (i, 0))],
            core_axis_name=("core", "subcore"),
            dimension_semantics=(pltpu.PARALLEL,),
        )(ids_hbm, out_hbm)

    return gather_kernel(table, ids2d)


@jax.jit
def kernel(token_ids, token_params, basin_proj_w):
    B, T = token_ids.shape
    table = _project_softmax_table(token_params, basin_proj_w)
    flat = token_ids.reshape(B * T).astype(jnp.int32)
    out = _sc_gather(table, flat)
    return out[:, :BASIN].reshape(B, T, BASIN)


# speedup vs baseline: 1.7885x; 1.0259x over previous
import jax
import jax.numpy as jnp
from jax.experimental import pallas as pl
from jax.experimental.pallas import tpu as pltpu
from jax.experimental.pallas import tpu_sc as plsc

VOCAB = 100000
HIDDEN = 128
BASIN = 64

_VOCAB_BLOCK = 4000
_GATHER_WINDOW = 256


def _proj_softmax_body(w_ref, tp_ref, out_ref):
    logits = jax.lax.dot_general(
        tp_ref[...], w_ref[...],
        (((1,), (1,)), ((), ())),
        preferred_element_type=jnp.float32,
    )
    m = jnp.max(logits, axis=-1, keepdims=True)
    e = jnp.exp(logits - m)
    sm = e / jnp.sum(e, axis=-1, keepdims=True)
    out_ref[:, :BASIN] = sm


def _project_softmax_table(token_params, basin_proj_w):
    grid = VOCAB // _VOCAB_BLOCK
    return pl.pallas_call(
        _proj_softmax_body,
        grid=(grid,),
        in_specs=[
            pl.BlockSpec((BASIN, HIDDEN), lambda i: (0, 0)),
            pl.BlockSpec((_VOCAB_BLOCK, HIDDEN), lambda i: (i, 0)),
        ],
        out_specs=pl.BlockSpec((_VOCAB_BLOCK, 2 * BASIN), lambda i: (i, 0)),
        out_shape=jax.ShapeDtypeStruct((VOCAB, 2 * BASIN), jnp.float32),
    )(basin_proj_w, token_params)


def _sc_gather(table, flat_ids):
    num_indices = flat_ids.shape[0]
    row_w = table.shape[1]
    ids2d = flat_ids.reshape(1, num_indices)
    mesh = plsc.VectorSubcoreMesh(core_axis_name="core",
                                  subcore_axis_name="subcore")

    @pl.kernel(
        out_type=jax.ShapeDtypeStruct((num_indices, row_w), jnp.float32),
        mesh=mesh,
    )
    def gather_kernel(table_hbm, ids_hbm, out_hbm):
        def body(ids_vmem, out_vmem):
            pltpu.sync_copy(table_hbm.at[ids_vmem.at[0]], out_vmem)

        pltpu.emit_pipeline(
            body,
            grid=(num_indices // _GATHER_WINDOW,),
            in_specs=[pl.BlockSpec((1, _GATHER_WINDOW),
                                   index_map=lambda i: (0, i))],
            out_specs=[pl.BlockSpec((_GATHER_WINDOW, row_w),
                                    index_map=lambda i: (i, 0))],
            core_axis_name=("core", "subcore"),
            dimension_semantics=(pltpu.PARALLEL,),
        )(ids_hbm, out_hbm)

    return gather_kernel(table, ids2d)


@jax.jit
def kernel(token_ids, token_params, basin_proj_w):
    B, T = token_ids.shape
    table = _project_softmax_table(token_params, basin_proj_w)
    flat = token_ids.reshape(B * T).astype(jnp.int32)
    n = B * T
    n_pad = ((n + _GATHER_WINDOW - 1) // _GATHER_WINDOW) * _GATHER_WINDOW
    if n_pad != n:
        flat = jnp.pad(flat, (0, n_pad - n))
    g = _sc_gather(table, flat)
    return g[:n, :BASIN].reshape(B, T, BASIN)
